# initial kernel scaffold (unmeasured)
import jax
import jax.numpy as jnp
from jax import lax
from jax.experimental import pallas as pl
from jax.experimental.pallas import tpu as pltpu

NZ = 4
SEQ = 1024
NH = 16
HD = 128
HALF = NH // 2
SCALE = HD ** -0.5


def _body(q_ref, k_hbm, v_hbm, out_ref,
          cw_ref, ccw_ref, l_ref,
          load_sems, cw_send, cw_recv, ccw_send, ccw_recv,
          cw_credit, ccw_credit):
    x = lax.axis_index("x")
    y = lax.axis_index("y")
    z = lax.axis_index("z")
    right = (x, y, lax.rem(z + 1, NZ))
    left = (x, y, lax.rem(z + NZ - 1, NZ))

    loads = [
        pltpu.make_async_copy(k_hbm.at[0:HALF], cw_ref.at[0, 0], load_sems.at[0]),
        pltpu.make_async_copy(v_hbm.at[0:HALF], cw_ref.at[0, 1], load_sems.at[1]),
        pltpu.make_async_copy(k_hbm.at[HALF:NH], ccw_ref.at[0, 0], load_sems.at[2]),
        pltpu.make_async_copy(v_hbm.at[HALF:NH], ccw_ref.at[0, 1], load_sems.at[3]),
    ]
    for ld in loads:
        ld.start()

    barrier = pltpu.get_barrier_semaphore()
    for nbr in (left, right):
        pl.semaphore_signal(barrier, inc=1, device_id=nbr,
                            device_id_type=pl.DeviceIdType.MESH)
    pl.semaphore_wait(barrier, 2)

    for ld in loads:
        ld.wait()

    def compute(buf_ref, slot, hbase, first):
        def head(i, _):
            h = hbase + i
            q = q_ref[h] * SCALE
            k = buf_ref[slot, 0, i]
            v = buf_ref[slot, 1, i]
            s = lax.dot_general(q, k, (((1,), (1,)), ((), ())),
                                preferred_element_type=jnp.float32)
            p = jnp.exp(s)
            pv = jnp.dot(p, v, preferred_element_type=jnp.float32)
            ls = jnp.sum(p, axis=1, keepdims=True)
            if first:
                out_ref[h] = pv
                l_ref[h] = ls
            else:
                out_ref[h] = out_ref[h] + pv
                l_ref[h] = l_ref[h] + ls
            return 0
        lax.fori_loop(0, HALF, head, 0)

    for h in range(NZ - 1):
        s = h % 2
        if h > 0:
            pl.semaphore_wait(cw_credit, 1)
            pl.semaphore_wait(ccw_credit, 1)
        rd_cw = pltpu.make_async_remote_copy(
            src_ref=cw_ref.at[s], dst_ref=cw_ref.at[1 - s],
            send_sem=cw_send.at[h], recv_sem=cw_recv.at[h],
            device_id=right, device_id_type=pl.DeviceIdType.MESH)
        rd_ccw = pltpu.make_async_remote_copy(
            src_ref=ccw_ref.at[s], dst_ref=ccw_ref.at[1 - s],
            send_sem=ccw_send.at[h], recv_sem=ccw_recv.at[h],
            device_id=left, device_id_type=pl.DeviceIdType.MESH)
        rd_cw.start()
        rd_ccw.start()
        compute(cw_ref, s, 0, h == 0)
        compute(ccw_ref, s, HALF, h == 0)
        rd_cw.wait_send()
        rd_ccw.wait_send()
        if h < NZ - 2:
            pl.semaphore_signal(cw_credit, inc=1, device_id=left,
                                device_id_type=pl.DeviceIdType.MESH)
            pl.semaphore_signal(ccw_credit, inc=1, device_id=right,
                                device_id_type=pl.DeviceIdType.MESH)
        rd_cw.wait_recv()
        rd_ccw.wait_recv()

    compute(cw_ref, 1, 0, False)
    compute(ccw_ref, 1, HALF, False)

    def norm(i, _):
        out_ref[i] = out_ref[i] / l_ref[i]
        return 0
    lax.fori_loop(0, NH, norm, 0)


def kernel(Q, K, V):
    qt = jnp.transpose(Q[0], (1, 0, 2))
    kt = jnp.transpose(K[0], (1, 0, 2))
    vt = jnp.transpose(V[0], (1, 0, 2))

    out_t = pl.pallas_call(
        _body,
        out_shape=jax.ShapeDtypeStruct((NH, SEQ, HD), jnp.float32),
        in_specs=[
            pl.BlockSpec(memory_space=pltpu.MemorySpace.VMEM),
            pl.BlockSpec(memory_space=pl.ANY),
            pl.BlockSpec(memory_space=pl.ANY),
        ],
        out_specs=pl.BlockSpec(memory_space=pltpu.MemorySpace.VMEM),
        scratch_shapes=[
            pltpu.VMEM((2, 2, HALF, SEQ, HD), jnp.float32),
            pltpu.VMEM((2, 2, HALF, SEQ, HD), jnp.float32),
            pltpu.VMEM((NH, SEQ, 1), jnp.float32),
            pltpu.SemaphoreType.DMA((4,)),
            pltpu.SemaphoreType.DMA((NZ - 1,)),
            pltpu.SemaphoreType.DMA((NZ - 1,)),
            pltpu.SemaphoreType.DMA((NZ - 1,)),
            pltpu.SemaphoreType.DMA((NZ - 1,)),
            pltpu.SemaphoreType.REGULAR,
            pltpu.SemaphoreType.REGULAR,
        ],
        compiler_params=pltpu.CompilerParams(collective_id=0),
    )(qt, kt, vt)

    return jnp.transpose(out_t, (1, 0, 2))[None]


# baseline (device time: 618859 ns/iter reference)
import jax
import jax.numpy as jnp
from jax import lax
from jax.experimental import pallas as pl
from jax.experimental.pallas import tpu as pltpu

NZ = 4
SEQ = 1024
NH = 16
HD = 128
HALF = NH // 2
SCALE = HD ** -0.5


def _body(q_ref, k_hbm, v_hbm, out_ref,
          cw_ref, ccw_ref, l_ref,
          load_sems, cw_send, cw_recv, ccw_send, ccw_recv,
          cw_credit, ccw_credit):
    x = lax.axis_index("x")
    y = lax.axis_index("y")
    z = lax.axis_index("z")
    right = (x, y, lax.rem(z + 1, NZ))
    left = (x, y, lax.rem(z + NZ - 1, NZ))

    loads = [
        pltpu.make_async_copy(k_hbm.at[0:HALF], cw_ref.at[0, 0], load_sems.at[0]),
        pltpu.make_async_copy(v_hbm.at[0:HALF], cw_ref.at[0, 1], load_sems.at[1]),
        pltpu.make_async_copy(k_hbm.at[HALF:NH], ccw_ref.at[0, 0], load_sems.at[2]),
        pltpu.make_async_copy(v_hbm.at[HALF:NH], ccw_ref.at[0, 1], load_sems.at[3]),
    ]
    for ld in loads:
        ld.start()

    barrier = pltpu.get_barrier_semaphore()
    for nbr in (left, right):
        pl.semaphore_signal(barrier, inc=1, device_id=nbr,
                            device_id_type=pl.DeviceIdType.MESH)
    pl.semaphore_wait(barrier, 2)

    for ld in loads:
        ld.wait()

    def compute(buf_ref, slot, hbase, first):
        def head(i, _):
            h = hbase + i
            q = q_ref[h] * SCALE
            k = buf_ref[slot, 0, i]
            v = buf_ref[slot, 1, i]
            s = lax.dot_general(q, k, (((1,), (1,)), ((), ())),
                                preferred_element_type=jnp.float32)
            p = jnp.exp(s)
            pv = jnp.dot(p, v, preferred_element_type=jnp.float32)
            ls = jnp.sum(p, axis=1, keepdims=True)
            if first:
                out_ref[h] = pv
                l_ref[h] = ls
            else:
                out_ref[h] = out_ref[h] + pv
                l_ref[h] = l_ref[h] + ls
            return 0
        lax.fori_loop(0, HALF, head, 0)

    for h in range(NZ - 1):
        s = h % 2
        if h > 0:
            pl.semaphore_wait(cw_credit, 1)
            pl.semaphore_wait(ccw_credit, 1)
        rd_cw = pltpu.make_async_remote_copy(
            src_ref=cw_ref.at[s], dst_ref=cw_ref.at[1 - s],
            send_sem=cw_send.at[h], recv_sem=cw_recv.at[h],
            device_id=right, device_id_type=pl.DeviceIdType.MESH)
        rd_ccw = pltpu.make_async_remote_copy(
            src_ref=ccw_ref.at[s], dst_ref=ccw_ref.at[1 - s],
            send_sem=ccw_send.at[h], recv_sem=ccw_recv.at[h],
            device_id=left, device_id_type=pl.DeviceIdType.MESH)
        rd_cw.start()
        rd_ccw.start()
        compute(cw_ref, s, 0, h == 0)
        compute(ccw_ref, s, HALF, h == 0)
        rd_cw.wait_send()
        rd_ccw.wait_send()
        if h < NZ - 2:
            pl.semaphore_signal(cw_credit, inc=1, device_id=left,
                                device_id_type=pl.DeviceIdType.MESH)
            pl.semaphore_signal(ccw_credit, inc=1, device_id=right,
                                device_id_type=pl.DeviceIdType.MESH)
        rd_cw.wait_recv()
        rd_ccw.wait_recv()

    compute(cw_ref, 1, 0, False)
    compute(ccw_ref, 1, HALF, False)

    def norm(i, _):
        out_ref[i] = out_ref[i] / l_ref[i]
        return 0
    lax.fori_loop(0, NH, norm, 0)


def kernel(Q, K, V):
    qt = jnp.transpose(Q[0], (1, 0, 2))
    kt = jnp.transpose(K[0], (1, 0, 2))
    vt = jnp.transpose(V[0], (1, 0, 2))

    out_t = pl.pallas_call(
        _body,
        out_shape=jax.ShapeDtypeStruct((NH, SEQ, HD), jnp.float32),
        in_specs=[
            pl.BlockSpec(memory_space=pltpu.MemorySpace.VMEM),
            pl.BlockSpec(memory_space=pl.ANY),
            pl.BlockSpec(memory_space=pl.ANY),
        ],
        out_specs=pl.BlockSpec(memory_space=pltpu.MemorySpace.VMEM),
        scratch_shapes=[
            pltpu.VMEM((2, 2, HALF, SEQ, HD), jnp.float32),
            pltpu.VMEM((2, 2, HALF, SEQ, HD), jnp.float32),
            pltpu.VMEM((NH, SEQ, 1), jnp.float32),
            pltpu.SemaphoreType.DMA((4,)),
            pltpu.SemaphoreType.DMA((NZ - 1,)),
            pltpu.SemaphoreType.DMA((NZ - 1,)),
            pltpu.SemaphoreType.DMA((NZ - 1,)),
            pltpu.SemaphoreType.DMA((NZ - 1,)),
            pltpu.SemaphoreType.REGULAR,
            pltpu.SemaphoreType.REGULAR,
        ],
        compiler_params=pltpu.CompilerParams(
            collective_id=0,
            vmem_limit_bytes=100 * 1024 * 1024,
        ),
    )(qt, kt, vt)

    return jnp.transpose(out_t, (1, 0, 2))[None]


# device time: 600081 ns/iter; 1.0313x vs baseline; 1.0313x over previous
import jax
import jax.numpy as jnp
from jax import lax
from jax.experimental import pallas as pl
from jax.experimental.pallas import tpu as pltpu

NZ = 4
SEQ = 1024
NH = 16
HD = 128
HALF = NH // 2
SUB = 4
HSUB = HALF // SUB
SCALE = HD ** -0.5


def _body(q_ref, k_hbm, v_hbm, out_ref,
          cw_ref, ccw_ref, l_ref,
          load_sems, cw_send, cw_recv, ccw_send, ccw_recv,
          cw_send2, cw_recv2, ccw_send2, ccw_recv2,
          cw_credit, ccw_credit):
    x = lax.axis_index("x")
    y = lax.axis_index("y")
    z = lax.axis_index("z")
    right = (x, y, lax.rem(z + 1, NZ))
    left = (x, y, lax.rem(z + NZ - 1, NZ))

    loads = []
    for u in range(SUB):
        lo, hi = u * HSUB, (u + 1) * HSUB
        loads += [
            pltpu.make_async_copy(k_hbm.at[lo:hi], cw_ref.at[0, u, 0],
                                  load_sems.at[len(loads)]),
            pltpu.make_async_copy(v_hbm.at[lo:hi], cw_ref.at[0, u, 1],
                                  load_sems.at[len(loads) + 1]),
            pltpu.make_async_copy(k_hbm.at[HALF + lo:HALF + hi],
                                  ccw_ref.at[0, u, 0],
                                  load_sems.at[len(loads) + 2]),
            pltpu.make_async_copy(v_hbm.at[HALF + lo:HALF + hi],
                                  ccw_ref.at[0, u, 1],
                                  load_sems.at[len(loads) + 3]),
        ]
    for ld in loads:
        ld.start()

    barrier = pltpu.get_barrier_semaphore()
    for nbr in (left, right):
        pl.semaphore_signal(barrier, inc=1, device_id=nbr,
                            device_id_type=pl.DeviceIdType.MESH)
    pl.semaphore_wait(barrier, 2)

    for ld in loads:
        ld.wait()

    def compute_sub(buf_ref, slot, u, hbase, first):
        def head(i, _):
            h = hbase + u * HSUB + i
            q = q_ref[h] * SCALE
            k = buf_ref[slot, u, 0, i]
            v = buf_ref[slot, u, 1, i]
            s = lax.dot_general(q, k, (((1,), (1,)), ((), ())),
                                preferred_element_type=jnp.float32)
            p = jnp.exp(s)
            pv = jnp.dot(p, v, preferred_element_type=jnp.float32)
            ls = jnp.sum(p, axis=1, keepdims=True)
            if first:
                out_ref[h] = pv
                l_ref[h] = ls
            else:
                out_ref[h] = out_ref[h] + pv
                l_ref[h] = l_ref[h] + ls
            return 0
        lax.fori_loop(0, HSUB, head, 0)

    def compute_slot(buf_ref, slot, hbase, first):
        for u in range(SUB):
            compute_sub(buf_ref, slot, u, hbase, first)

    for h in range(NZ - 2):
        s = h % 2
        if h > 0:
            pl.semaphore_wait(cw_credit, 1)
            pl.semaphore_wait(ccw_credit, 1)
        rd_cw = pltpu.make_async_remote_copy(
            src_ref=cw_ref.at[s], dst_ref=cw_ref.at[1 - s],
            send_sem=cw_send.at[h], recv_sem=cw_recv.at[h],
            device_id=right, device_id_type=pl.DeviceIdType.MESH)
        rd_ccw = pltpu.make_async_remote_copy(
            src_ref=ccw_ref.at[s], dst_ref=ccw_ref.at[1 - s],
            send_sem=ccw_send.at[h], recv_sem=ccw_recv.at[h],
            device_id=left, device_id_type=pl.DeviceIdType.MESH)
        rd_cw.start()
        rd_ccw.start()
        compute_slot(cw_ref, s, 0, h == 0)
        compute_slot(ccw_ref, s, HALF, h == 0)
        rd_cw.wait_send()
        rd_ccw.wait_send()
        pl.semaphore_signal(cw_credit, inc=1, device_id=left,
                            device_id_type=pl.DeviceIdType.MESH)
        pl.semaphore_signal(ccw_credit, inc=1, device_id=right,
                            device_id_type=pl.DeviceIdType.MESH)
        rd_cw.wait_recv()
        rd_ccw.wait_recv()

    pl.semaphore_wait(cw_credit, 1)
    pl.semaphore_wait(ccw_credit, 1)
    subs = []
    for u in range(SUB):
        sd_cw = pltpu.make_async_remote_copy(
            src_ref=cw_ref.at[0, u], dst_ref=cw_ref.at[1, u],
            send_sem=cw_send2.at[u], recv_sem=cw_recv2.at[u],
            device_id=right, device_id_type=pl.DeviceIdType.MESH)
        sd_ccw = pltpu.make_async_remote_copy(
            src_ref=ccw_ref.at[0, u], dst_ref=ccw_ref.at[1, u],
            send_sem=ccw_send2.at[u], recv_sem=ccw_recv2.at[u],
            device_id=left, device_id_type=pl.DeviceIdType.MESH)
        sd_cw.start()
        sd_ccw.start()
        subs.append((sd_cw, sd_ccw))
    compute_slot(cw_ref, 0, 0, False)
    compute_slot(ccw_ref, 0, HALF, False)
    for u in range(SUB):
        sd_cw, sd_ccw = subs[u]
        sd_cw.wait_recv()
        compute_sub(cw_ref, 1, u, 0, False)
        sd_ccw.wait_recv()
        compute_sub(ccw_ref, 1, u, HALF, False)
    for sd_cw, sd_ccw in subs:
        sd_cw.wait_send()
        sd_ccw.wait_send()

    def norm(i, _):
        out_ref[i] = out_ref[i] / l_ref[i]
        return 0
    lax.fori_loop(0, NH, norm, 0)


def kernel(Q, K, V):
    qt = jnp.transpose(Q[0], (1, 0, 2))
    kt = jnp.transpose(K[0], (1, 0, 2))
    vt = jnp.transpose(V[0], (1, 0, 2))

    out_t = pl.pallas_call(
        _body,
        out_shape=jax.ShapeDtypeStruct((NH, SEQ, HD), jnp.float32),
        in_specs=[
            pl.BlockSpec(memory_space=pltpu.MemorySpace.VMEM),
            pl.BlockSpec(memory_space=pl.ANY),
            pl.BlockSpec(memory_space=pl.ANY),
        ],
        out_specs=pl.BlockSpec(memory_space=pltpu.MemorySpace.VMEM),
        scratch_shapes=[
            pltpu.VMEM((2, SUB, 2, HSUB, SEQ, HD), jnp.float32),
            pltpu.VMEM((2, SUB, 2, HSUB, SEQ, HD), jnp.float32),
            pltpu.VMEM((NH, SEQ, 1), jnp.float32),
            pltpu.SemaphoreType.DMA((4 * SUB,)),
            pltpu.SemaphoreType.DMA((NZ - 2,)),
            pltpu.SemaphoreType.DMA((NZ - 2,)),
            pltpu.SemaphoreType.DMA((NZ - 2,)),
            pltpu.SemaphoreType.DMA((NZ - 2,)),
            pltpu.SemaphoreType.DMA((SUB,)),
            pltpu.SemaphoreType.DMA((SUB,)),
            pltpu.SemaphoreType.DMA((SUB,)),
            pltpu.SemaphoreType.DMA((SUB,)),
            pltpu.SemaphoreType.REGULAR,
            pltpu.SemaphoreType.REGULAR,
        ],
        compiler_params=pltpu.CompilerParams(
            collective_id=0,
            vmem_limit_bytes=100 * 1024 * 1024,
        ),
    )(qt, kt, vt)

    return jnp.transpose(out_t, (1, 0, 2))[None]


# device time: 326784 ns/iter; 1.8938x vs baseline; 1.8363x over previous
import jax
import jax.numpy as jnp
from jax import lax
from jax.experimental import pallas as pl
from jax.experimental.pallas import tpu as pltpu

NZ = 4
SEQ = 1024
NH = 16
HD = 128
HALF = NH // 2
SUB = 4
HSUB = HALF // SUB
SCALE = HD ** -0.5


def _body(q_ref, k_hbm, v_hbm, out_ref,
          cw_ref, ccw_ref, l_ref,
          load_sems, cw_send, cw_recv, ccw_send, ccw_recv,
          cw_send2, cw_recv2, ccw_send2, ccw_recv2,
          cw_credit, ccw_credit):
    x = lax.axis_index("x")
    y = lax.axis_index("y")
    z = lax.axis_index("z")
    right = (x, y, lax.rem(z + 1, NZ))
    left = (x, y, lax.rem(z + NZ - 1, NZ))

    loads = []
    for u in range(SUB):
        lo, hi = u * HSUB, (u + 1) * HSUB
        loads += [
            pltpu.make_async_copy(k_hbm.at[lo:hi], cw_ref.at[0, u, 0],
                                  load_sems.at[len(loads)]),
            pltpu.make_async_copy(v_hbm.at[lo:hi], cw_ref.at[0, u, 1],
                                  load_sems.at[len(loads) + 1]),
            pltpu.make_async_copy(k_hbm.at[HALF + lo:HALF + hi],
                                  ccw_ref.at[0, u, 0],
                                  load_sems.at[len(loads) + 2]),
            pltpu.make_async_copy(v_hbm.at[HALF + lo:HALF + hi],
                                  ccw_ref.at[0, u, 1],
                                  load_sems.at[len(loads) + 3]),
        ]
    for ld in loads:
        ld.start()

    barrier = pltpu.get_barrier_semaphore()
    for nbr in (left, right):
        pl.semaphore_signal(barrier, inc=1, device_id=nbr,
                            device_id_type=pl.DeviceIdType.MESH)
    pl.semaphore_wait(barrier, 2)

    for ld in loads:
        ld.wait()

    def compute_sub(buf_ref, slot, u, hbase, first):
        def head(i, _):
            h = hbase + u * HSUB + i
            q = (q_ref[h] * SCALE).astype(jnp.bfloat16)
            k = buf_ref[slot, u, 0, i]
            v = buf_ref[slot, u, 1, i]
            s = lax.dot_general(q, k, (((1,), (1,)), ((), ())),
                                preferred_element_type=jnp.float32)
            p = jnp.exp(s)
            pv = jnp.dot(p.astype(jnp.bfloat16), v,
                         preferred_element_type=jnp.float32)
            ls = jnp.sum(p, axis=1, keepdims=True)
            if first:
                out_ref[h] = pv
                l_ref[h] = ls
            else:
                out_ref[h] = out_ref[h] + pv
                l_ref[h] = l_ref[h] + ls
            return 0
        lax.fori_loop(0, HSUB, head, 0)

    def compute_slot(buf_ref, slot, hbase, first):
        for u in range(SUB):
            compute_sub(buf_ref, slot, u, hbase, first)

    for h in range(NZ - 2):
        s = h % 2
        if h > 0:
            pl.semaphore_wait(cw_credit, 1)
            pl.semaphore_wait(ccw_credit, 1)
        rd_cw = pltpu.make_async_remote_copy(
            src_ref=cw_ref.at[s], dst_ref=cw_ref.at[1 - s],
            send_sem=cw_send.at[h], recv_sem=cw_recv.at[h],
            device_id=right, device_id_type=pl.DeviceIdType.MESH)
        rd_ccw = pltpu.make_async_remote_copy(
            src_ref=ccw_ref.at[s], dst_ref=ccw_ref.at[1 - s],
            send_sem=ccw_send.at[h], recv_sem=ccw_recv.at[h],
            device_id=left, device_id_type=pl.DeviceIdType.MESH)
        rd_cw.start()
        rd_ccw.start()
        compute_slot(cw_ref, s, 0, h == 0)
        compute_slot(ccw_ref, s, HALF, h == 0)
        rd_cw.wait_send()
        rd_ccw.wait_send()
        pl.semaphore_signal(cw_credit, inc=1, device_id=left,
                            device_id_type=pl.DeviceIdType.MESH)
        pl.semaphore_signal(ccw_credit, inc=1, device_id=right,
                            device_id_type=pl.DeviceIdType.MESH)
        rd_cw.wait_recv()
        rd_ccw.wait_recv()

    pl.semaphore_wait(cw_credit, 1)
    pl.semaphore_wait(ccw_credit, 1)
    subs = []
    for u in range(SUB):
        sd_cw = pltpu.make_async_remote_copy(
            src_ref=cw_ref.at[0, u], dst_ref=cw_ref.at[1, u],
            send_sem=cw_send2.at[u], recv_sem=cw_recv2.at[u],
            device_id=right, device_id_type=pl.DeviceIdType.MESH)
        sd_ccw = pltpu.make_async_remote_copy(
            src_ref=ccw_ref.at[0, u], dst_ref=ccw_ref.at[1, u],
            send_sem=ccw_send2.at[u], recv_sem=ccw_recv2.at[u],
            device_id=left, device_id_type=pl.DeviceIdType.MESH)
        sd_cw.start()
        sd_ccw.start()
        subs.append((sd_cw, sd_ccw))
    compute_slot(cw_ref, 0, 0, False)
    compute_slot(ccw_ref, 0, HALF, False)
    for u in range(SUB):
        sd_cw, sd_ccw = subs[u]
        sd_cw.wait_recv()
        compute_sub(cw_ref, 1, u, 0, False)
        sd_ccw.wait_recv()
        compute_sub(ccw_ref, 1, u, HALF, False)
    for sd_cw, sd_ccw in subs:
        sd_cw.wait_send()
        sd_ccw.wait_send()

    def norm(i, _):
        out_ref[i] = out_ref[i] / l_ref[i]
        return 0
    lax.fori_loop(0, NH, norm, 0)


def kernel(Q, K, V):
    qt = jnp.transpose(Q[0], (1, 0, 2))
    kt = jnp.transpose(K[0], (1, 0, 2)).astype(jnp.bfloat16)
    vt = jnp.transpose(V[0], (1, 0, 2)).astype(jnp.bfloat16)

    out_t = pl.pallas_call(
        _body,
        out_shape=jax.ShapeDtypeStruct((NH, SEQ, HD), jnp.float32),
        in_specs=[
            pl.BlockSpec(memory_space=pltpu.MemorySpace.VMEM),
            pl.BlockSpec(memory_space=pl.ANY),
            pl.BlockSpec(memory_space=pl.ANY),
        ],
        out_specs=pl.BlockSpec(memory_space=pltpu.MemorySpace.VMEM),
        scratch_shapes=[
            pltpu.VMEM((2, SUB, 2, HSUB, SEQ, HD), jnp.bfloat16),
            pltpu.VMEM((2, SUB, 2, HSUB, SEQ, HD), jnp.bfloat16),
            pltpu.VMEM((NH, SEQ, 1), jnp.float32),
            pltpu.SemaphoreType.DMA((4 * SUB,)),
            pltpu.SemaphoreType.DMA((NZ - 2,)),
            pltpu.SemaphoreType.DMA((NZ - 2,)),
            pltpu.SemaphoreType.DMA((NZ - 2,)),
            pltpu.SemaphoreType.DMA((NZ - 2,)),
            pltpu.SemaphoreType.DMA((SUB,)),
            pltpu.SemaphoreType.DMA((SUB,)),
            pltpu.SemaphoreType.DMA((SUB,)),
            pltpu.SemaphoreType.DMA((SUB,)),
            pltpu.SemaphoreType.REGULAR,
            pltpu.SemaphoreType.REGULAR,
        ],
        compiler_params=pltpu.CompilerParams(
            collective_id=0,
            vmem_limit_bytes=100 * 1024 * 1024,
        ),
    )(qt, kt, vt)

    return jnp.transpose(out_t, (1, 0, 2))[None]
